# baseline (speedup 1.0000x reference)
import functools

import jax
import jax.numpy as jnp
from jax import lax
from jax.experimental import pallas as pl
from jax.experimental.pallas import tpu as pltpu
from jax.experimental.pallas import tpu_sc as plsc

_NUM_CORES = 2
_NUM_SUBCORES = 16
_NW = _NUM_CORES * _NUM_SUBCORES
_LANES = 16


@functools.partial(jax.jit, static_argnums=(3, 4, 5, 6, 7))
def _embed_add(ids_pad, token_embedding, ctx_flat, B, W, WP, V, D):
    rows_per_w = B // _NW
    mesh = plsc.VectorSubcoreMesh(core_axis_name="c", subcore_axis_name="s")

    @functools.partial(
        pl.kernel,
        out_type=jax.ShapeDtypeStruct((B, WP, D), jnp.float32),
        mesh=mesh,
        scratch_types=[
            pltpu.VMEM((WP,), jnp.int32),
            pltpu.VMEM((WP, D), jnp.float32),
            pltpu.VMEM((W * D,), jnp.float32),
            pltpu.SemaphoreType.DMA,
        ],
    )
    def k(ids_hbm, table_hbm, ctx_hbm, out_hbm, idx_row, rows_v, ctx_v, sem):
        wid = lax.axis_index("s") * _NUM_CORES + lax.axis_index("c")
        base = wid * rows_per_w
        pltpu.sync_copy(ctx_hbm, ctx_v)

        def per_row(r, carry):
            b = base + r
            pltpu.sync_copy(ids_hbm.at[pl.ds(b * WP, WP)], idx_row)
            pltpu.async_copy(table_hbm.at[idx_row], rows_v, sem).wait()

            def per_word(w, carry2):
                cbase = w * D
                for j in range(D // _LANES):
                    sl = pl.ds(j * _LANES, _LANES)
                    plsc.addupdate(rows_v.at[w, sl],
                                   ctx_v[pl.ds(cbase + j * _LANES, _LANES)])
                return carry2

            lax.fori_loop(0, W, per_word, 0)

            pltpu.sync_copy(rows_v, out_hbm.at[b])
            return carry

        lax.fori_loop(0, rows_per_w, per_row, 0)

    return k(ids_pad, token_embedding, ctx_flat)


def kernel(prompt_ids, token_embedding, ctx):
    B, W = prompt_ids.shape
    V, D = token_embedding.shape
    WP = (W + 7) // 8 * 8
    ids = prompt_ids.astype(jnp.int32)
    ids_pad = jnp.pad(ids, ((0, 0), (0, WP - W))).reshape(-1)
    out = _embed_add(ids_pad, token_embedding, ctx.reshape(-1), B, W, WP, V, D)
    return out[:, :W, :]
